# baseline (speedup 1.0000x reference)
import jax
import jax.numpy as jnp
from jax import lax
from jax.experimental import pallas as pl
from jax.experimental.pallas import tpu as pltpu
from jax.experimental.pallas import tpu_sc as plsc

M = 100000
D = 128
B = 16384
NC = 2
NS = 16
NW = NC * NS

RPW = 3136
LAST_ROWS = M - (NW - 1) * RPW
PT = RPW
BPW = B // NW
GCH = 128
SCH = 128
NCH_MAX = (RPW + SCH - 1) // SCH
LIST_CAP = RPW + SCH


def _body(mem, values, ts, ids, gathered, new_mem, new_lu,
          ids_v, ts_v, last_b, lu_v, winb_flat, dstr_flat, dstr2d,
          gbuf_a, gbuf_b, sbuf, sem_t, sem_c, sem_g, sem_s):
    wid = lax.axis_index("s") * NC + lax.axis_index("c")
    base_r = wid * RPW
    is_last = wid == NW - 1
    nrows = jnp.where(is_last, LAST_ROWS, RPW).astype(jnp.int32)
    nvec = nrows // 16
    iota = lax.broadcasted_iota(jnp.int32, (16,), 0)

    pltpu.sync_copy(ids, ids_v)
    cp_ts = pltpu.async_copy(ts, ts_v, sem_t)

    @pl.when(jnp.logical_not(is_last))
    def _():
        pltpu.async_copy(mem.at[pl.ds(base_r, RPW)],
                         new_mem.at[pl.ds(base_r, RPW)], sem_c)

    @pl.when(is_last)
    def _():
        pltpu.async_copy(mem.at[pl.ds(base_r, LAST_ROWS)],
                         new_mem.at[pl.ds(base_r, LAST_ROWS)], sem_c)

    gbase = wid * BPW
    g0 = pltpu.async_copy(mem.at[ids_v.at[pl.ds(gbase, GCH)]], gbuf_a, sem_g)

    def init_body(i, _):
        last_b[pl.ds(i * 16, 16)] = jnp.full((16,), -1, jnp.int32)
        return 0
    lax.fori_loop(0, PT // 16, init_body, 0)

    def scan_body(i, _):
        ids16 = ids_v[pl.ds(i * 16, 16)]
        mine = (ids16 >= base_r) & (ids16 < base_r + nrows)
        nmine = jnp.sum(mine.astype(jnp.int32))

        @pl.when(nmine > 0)
        def _():
            loc = ids16 - base_r
            bvec = i * 16 + iota
            plsc.store_scatter(last_b, [loc], bvec, mask=mine)
            rb = plsc.load_gather(last_b, [loc], mask=mine)
            need = mine & (rb < bvec)
            nf = jnp.sum(need.astype(jnp.int32))

            def fix_body(c):
                rb2 = plsc.load_gather(last_b, [loc], mask=mine)
                need2 = mine & (rb2 < bvec)
                plsc.store_scatter(last_b, [loc], bvec, mask=need2)
                rb3 = plsc.load_gather(last_b, [loc], mask=mine)
                need3 = mine & (rb3 < bvec)
                return jnp.sum(need3.astype(jnp.int32))

            lax.while_loop(lambda c: c > 0, fix_body, nf)
        return 0
    lax.fori_loop(0, B // 16, scan_body, 0)

    g0.wait()
    g1 = pltpu.async_copy(mem.at[ids_v.at[pl.ds(gbase + GCH, GCH)]], gbuf_b, sem_g)
    pltpu.sync_copy(gbuf_a, gathered.at[pl.ds(gbase, GCH)])
    g1.wait()
    g2 = pltpu.async_copy(mem.at[ids_v.at[pl.ds(gbase + 2 * GCH, GCH)]], gbuf_a, sem_g)
    pltpu.sync_copy(gbuf_b, gathered.at[pl.ds(gbase + GCH, GCH)])
    g2.wait()
    g3 = pltpu.async_copy(mem.at[ids_v.at[pl.ds(gbase + 3 * GCH, GCH)]], gbuf_b, sem_g)
    pltpu.sync_copy(gbuf_a, gathered.at[pl.ds(gbase + 2 * GCH, GCH)])
    g3.wait()
    pltpu.sync_copy(gbuf_b, gathered.at[pl.ds(gbase + 3 * GCH, GCH)])

    cp_ts.wait()

    def tbl_body(i, cnt):
        lb = last_b[pl.ds(i * 16, 16)]
        m = lb >= 0
        t = plsc.load_gather(ts_v, [lb], mask=m)
        lu_v[pl.ds(i * 16, 16)] = jnp.where(m, t, jnp.float32(0.0))
        plsc.store_compressed(winb_flat.at[pl.ds(cnt, 16)], lb, mask=m)
        grow = base_r + i * 16 + iota
        plsc.store_compressed(dstr_flat.at[pl.ds(cnt, 16)], grow, mask=m)
        return cnt + jnp.sum(m.astype(jnp.int32))
    cnt = lax.fori_loop(0, nvec, tbl_body, jnp.int32(0))

    @pl.when(cnt > 0)
    def _():
        lastix = jnp.full((16,), cnt - 1, jnp.int32)
        wpad = plsc.load_gather(winb_flat, [lastix])
        dpad = plsc.load_gather(dstr_flat, [lastix])
        for k in range(SCH // 16):
            winb_flat[pl.ds(cnt + k * 16, 16)] = wpad
            dstr_flat[pl.ds(cnt + k * 16, 16)] = dpad

    nch = (cnt + SCH - 1) // SCH

    def tr_body(j, _):
        v = dstr_flat[pl.ds(j * 16, 16)]
        dstr2d[j // 8, pl.ds((j % 8) * 16, 16)] = v
        return 0
    lax.fori_loop(0, nch * (SCH // 16), tr_body, 0)

    @pl.when(jnp.logical_not(is_last))
    def _():
        pltpu.sync_copy(lu_v.at[pl.ds(0, RPW)], new_lu.at[pl.ds(base_r, RPW)])

    @pl.when(is_last)
    def _():
        pltpu.sync_copy(lu_v.at[pl.ds(0, LAST_ROWS)],
                        new_lu.at[pl.ds(base_r, LAST_ROWS)])

    @pl.when(jnp.logical_not(is_last))
    def _():
        pltpu.make_async_copy(mem.at[pl.ds(base_r, RPW)],
                              new_mem.at[pl.ds(base_r, RPW)], sem_c).wait()

    @pl.when(is_last)
    def _():
        pltpu.make_async_copy(mem.at[pl.ds(base_r, LAST_ROWS)],
                              new_mem.at[pl.ds(base_r, LAST_ROWS)], sem_c).wait()

    def sc_body(c, _):
        pltpu.async_copy(values.at[winb_flat.at[pl.ds(c * SCH, SCH)]],
                         sbuf, sem_s).wait()
        pltpu.async_copy(sbuf, new_mem.at[dstr2d.at[c]], sem_s).wait()
        return 0
    lax.fori_loop(0, nch, sc_body, 0)


def kernel(mem, values, timestamps, node_ids):
    mesh = plsc.VectorSubcoreMesh(core_axis_name="c", subcore_axis_name="s")
    out = pl.kernel(
        _body,
        out_type=(
            jax.ShapeDtypeStruct((B, D), jnp.float32),
            jax.ShapeDtypeStruct((M, D), jnp.float32),
            jax.ShapeDtypeStruct((M,), jnp.float32),
        ),
        mesh=mesh,
        compiler_params=pltpu.CompilerParams(needs_layout_passes=False),
        scratch_types=[
            pltpu.VMEM((B,), jnp.int32),
            pltpu.VMEM((B,), jnp.float32),
            pltpu.VMEM((PT,), jnp.int32),
            pltpu.VMEM((PT,), jnp.float32),
            pltpu.VMEM((LIST_CAP,), jnp.int32),
            pltpu.VMEM((LIST_CAP,), jnp.int32),
            pltpu.VMEM((NCH_MAX, SCH), jnp.int32),
            pltpu.VMEM((GCH, D), jnp.float32),
            pltpu.VMEM((GCH, D), jnp.float32),
            pltpu.VMEM((SCH, D), jnp.float32),
            pltpu.SemaphoreType.DMA,
            pltpu.SemaphoreType.DMA,
            pltpu.SemaphoreType.DMA,
            pltpu.SemaphoreType.DMA,
        ],
    )(mem, values, timestamps, node_ids)
    return out


# speedup vs baseline: 12.3870x; 12.3870x over previous
import jax
import jax.numpy as jnp
from jax import lax
from jax.experimental import pallas as pl
from jax.experimental.pallas import tpu as pltpu
from jax.experimental.pallas import tpu_sc as plsc

M = 100000
D = 128
B = 16384
NC = 2
NS = 16
NW = NC * NS

RPW = 3136
LAST_ROWS = M - (NW - 1) * RPW
PT = RPW
BPW = B // NW
GCH = 128
SCH = 128
CCH = 128
CCT = 32
NCH_MAX = (RPW + SCH - 1) // SCH
LIST_CAP = RPW + 2 * SCH


def _body(mem, values, ts, ids, gathered, new_mem, new_lu,
          ids_v, ts_v, last_b, lu_v, winb_flat, dstr_flat, dstr2d,
          gbuf, sbuf, cb0, cb1,
          sem_t, sem_g, sem_s, sg0, sg1, ss0, ss1):
    wid = lax.axis_index("s") * NC + lax.axis_index("c")
    base_r = wid * RPW
    is_last = wid == NW - 1
    nrows = jnp.where(is_last, LAST_ROWS, RPW).astype(jnp.int32)
    nvec = nrows // 16
    iota = lax.broadcasted_iota(jnp.int32, (16,), 0)

    pltpu.sync_copy(ids, ids_v)
    cp_ts = pltpu.async_copy(ts, ts_v, sem_t)

    ncc = nrows // CCH
    ntail = (nrows - ncc * CCH) // CCT

    def cgather(c, buf, sem):
        pltpu.async_copy(mem.at[pl.ds(base_r + c * CCH, CCH)], buf, sem)

    def cscatter(c, buf, sem):
        pltpu.async_copy(buf, new_mem.at[pl.ds(base_r + c * CCH, CCH)], sem)

    def cwait_g(c, buf, sem):
        pltpu.make_async_copy(mem.at[pl.ds(base_r + c * CCH, CCH)], buf, sem).wait()

    def cwait_s(c, buf, sem):
        pltpu.make_async_copy(buf, new_mem.at[pl.ds(base_r + c * CCH, CCH)],
                              sem).wait()

    cgather(0, cb0, sg0)

    def copy_body(c, _):
        def stage(buf_c, buf_n, sg_c, sg_n, ss_c, ss_n):
            @pl.when(c + 1 < ncc)
            def _():
                @pl.when(c >= 1)
                def _():
                    cwait_s(c - 1, buf_n, ss_n)
                cgather(c + 1, buf_n, sg_n)
            cwait_g(c, buf_c, sg_c)
            cscatter(c, buf_c, ss_c)

        @pl.when(c % 2 == 0)
        def _():
            stage(cb0, cb1, sg0, sg1, ss0, ss1)

        @pl.when(c % 2 == 1)
        def _():
            stage(cb1, cb0, sg1, sg0, ss1, ss0)
        return 0
    lax.fori_loop(0, ncc, copy_body, 0)

    @pl.when(ncc % 2 == 0)
    def _():
        cwait_s(ncc - 2, cb0, ss0)
        cwait_s(ncc - 1, cb1, ss1)

    @pl.when(ncc % 2 == 1)
    def _():
        cwait_s(ncc - 2, cb1, ss1)
        cwait_s(ncc - 1, cb0, ss0)

    tbase = base_r + ncc * CCH

    def tail_body(t, _):
        pltpu.async_copy(mem.at[pl.ds(tbase + t * CCT, CCT)],
                         cb0.at[pl.ds(0, CCT)], sg0).wait()
        pltpu.async_copy(cb0.at[pl.ds(0, CCT)],
                         new_mem.at[pl.ds(tbase + t * CCT, CCT)], ss0).wait()
        return 0
    lax.fori_loop(0, ntail, tail_body, 0)

    def init_body(i, _):
        last_b[pl.ds(i * 16, 16)] = jnp.full((16,), -1, jnp.int32)
        return 0
    lax.fori_loop(0, PT // 16, init_body, 0)

    def scan_body(i, _):
        ids16 = ids_v[pl.ds(i * 16, 16)]
        mine = (ids16 >= base_r) & (ids16 < base_r + nrows)
        loc = ids16 - base_r
        bvec = i * 16 + iota
        plsc.store_scatter(last_b, [loc], bvec, mask=mine)
        rb = plsc.load_gather(last_b, [loc], mask=mine)
        need = mine & (rb < bvec)
        nf = jnp.sum(need.astype(jnp.int32))

        @pl.when(nf > 0)
        def _():
            def fix_body(c):
                rb2 = plsc.load_gather(last_b, [loc], mask=mine)
                need2 = mine & (rb2 < bvec)
                plsc.store_scatter(last_b, [loc], bvec, mask=need2)
                rb3 = plsc.load_gather(last_b, [loc], mask=mine)
                need3 = mine & (rb3 < bvec)
                return jnp.sum(need3.astype(jnp.int32))
            lax.while_loop(lambda c: c > 0, fix_body, nf)
        return 0

    gbase = wid * BPW
    SLAB = B // 16 // 4
    for k in range(4):
        gk = pltpu.async_copy(mem.at[ids_v.at[pl.ds(gbase + k * GCH, GCH)]],
                              gbuf, sem_g)
        lax.fori_loop(k * SLAB, (k + 1) * SLAB, scan_body, 0)
        gk.wait()
        pltpu.sync_copy(gbuf, gathered.at[pl.ds(gbase + k * GCH, GCH)])

    cp_ts.wait()

    def tbl_body(i, cnt):
        lb = last_b[pl.ds(i * 16, 16)]
        m = lb >= 0
        t = plsc.load_gather(ts_v, [lb], mask=m)
        lu_v[pl.ds(i * 16, 16)] = jnp.where(m, t, jnp.float32(0.0))
        plsc.store_compressed(winb_flat.at[pl.ds(cnt, 16)], lb, mask=m)
        grow = base_r + i * 16 + iota
        plsc.store_compressed(dstr_flat.at[pl.ds(cnt, 16)], grow, mask=m)
        return cnt + jnp.sum(m.astype(jnp.int32))
    cnt = lax.fori_loop(0, nvec, tbl_body, jnp.int32(0))

    @pl.when(cnt > 0)
    def _():
        lastix = jnp.full((16,), cnt - 1, jnp.int32)
        wpad = plsc.load_gather(winb_flat, [lastix])
        dpad = plsc.load_gather(dstr_flat, [lastix])
        for k in range(SCH // 16):
            winb_flat[pl.ds(cnt + k * 16, 16)] = wpad
            dstr_flat[pl.ds(cnt + k * 16, 16)] = dpad

    nch = (cnt + SCH - 1) // SCH

    def tr_body(j, _):
        v = dstr_flat[pl.ds(j * 16, 16)]
        dstr2d[j // 8, pl.ds((j % 8) * 16, 16)] = v
        return 0
    lax.fori_loop(0, nch * (SCH // 16), tr_body, 0)

    @pl.when(jnp.logical_not(is_last))
    def _():
        pltpu.sync_copy(lu_v.at[pl.ds(0, RPW)], new_lu.at[pl.ds(base_r, RPW)])

    @pl.when(is_last)
    def _():
        pltpu.sync_copy(lu_v.at[pl.ds(0, LAST_ROWS)],
                        new_lu.at[pl.ds(base_r, LAST_ROWS)])

    def sc_body(c, _):
        pltpu.async_copy(values.at[winb_flat.at[pl.ds(c * SCH, SCH)]],
                         sbuf, sem_s).wait()
        pltpu.async_copy(sbuf, new_mem.at[dstr2d.at[c]], sem_s).wait()
        return 0
    lax.fori_loop(0, nch, sc_body, 0)


def kernel(mem, values, timestamps, node_ids):
    mesh = plsc.VectorSubcoreMesh(core_axis_name="c", subcore_axis_name="s")
    out = pl.kernel(
        _body,
        out_type=(
            jax.ShapeDtypeStruct((B, D), jnp.float32),
            jax.ShapeDtypeStruct((M, D), jnp.float32),
            jax.ShapeDtypeStruct((M,), jnp.float32),
        ),
        mesh=mesh,
        compiler_params=pltpu.CompilerParams(needs_layout_passes=False),
        scratch_types=[
            pltpu.VMEM((B,), jnp.int32),
            pltpu.VMEM((B,), jnp.float32),
            pltpu.VMEM((PT,), jnp.int32),
            pltpu.VMEM((PT,), jnp.float32),
            pltpu.VMEM((LIST_CAP,), jnp.int32),
            pltpu.VMEM((LIST_CAP,), jnp.int32),
            pltpu.VMEM((NCH_MAX, SCH), jnp.int32),
            pltpu.VMEM((GCH, D), jnp.float32),
            pltpu.VMEM((SCH, D), jnp.float32),
            pltpu.VMEM((CCH, D), jnp.float32),
            pltpu.VMEM((CCH, D), jnp.float32),
            pltpu.SemaphoreType.DMA,
            pltpu.SemaphoreType.DMA,
            pltpu.SemaphoreType.DMA,
            pltpu.SemaphoreType.DMA,
            pltpu.SemaphoreType.DMA,
            pltpu.SemaphoreType.DMA,
            pltpu.SemaphoreType.DMA,
        ],
    )(mem, values, timestamps, node_ids)
    return out


# speedup vs baseline: 12.6194x; 1.0188x over previous
import jax
import jax.numpy as jnp
from jax import lax
from jax.experimental import pallas as pl
from jax.experimental.pallas import tpu as pltpu
from jax.experimental.pallas import tpu_sc as plsc

M = 100000
D = 128
B = 16384
NC = 2
NS = 16
NW = NC * NS

RPW = 3136
LAST_ROWS = M - (NW - 1) * RPW
PT = RPW
BPW = B // NW
GCH = 128
SCH = 128
CCH = 128
CCT = 32
NCH_MAX = (RPW + SCH - 1) // SCH
LIST_CAP = RPW + 2 * SCH


def _body(mem, values, ts, ids, gathered, new_mem, new_lu,
          ids_v, ts_v, last_b, lu_v, winb_flat, dstr_flat, dstr2d,
          gbuf, sbuf, cb0, cb1,
          sem_t, sem_g, sem_s, sg0, sg1, ss0, ss1):
    wid = lax.axis_index("s") * NC + lax.axis_index("c")
    base_r = wid * RPW
    is_last = wid == NW - 1
    nrows = jnp.where(is_last, LAST_ROWS, RPW).astype(jnp.int32)
    nvec = nrows // 16
    iota = lax.broadcasted_iota(jnp.int32, (16,), 0)

    with jax.named_scope("p0_stage"):
        pltpu.sync_copy(ids, ids_v)
        cp_ts = pltpu.async_copy(ts, ts_v, sem_t)

    ncc = nrows // CCH
    ntail = (nrows - ncc * CCH) // CCT

    def cgather(c, buf, sem):
        pltpu.async_copy(mem.at[pl.ds(base_r + c * CCH, CCH)], buf, sem)

    def cscatter(c, buf, sem):
        pltpu.async_copy(buf, new_mem.at[pl.ds(base_r + c * CCH, CCH)], sem)

    def cwait_g(c, buf, sem):
        pltpu.make_async_copy(mem.at[pl.ds(base_r + c * CCH, CCH)], buf, sem).wait()

    def cwait_s(c, buf, sem):
        pltpu.make_async_copy(buf, new_mem.at[pl.ds(base_r + c * CCH, CCH)],
                              sem).wait()

    cgather(0, cb0, sg0)

    def copy_body(c, _):
        def stage(buf_c, buf_n, sg_c, sg_n, ss_c, ss_n):
            @pl.when(c + 1 < ncc)
            def _():
                @pl.when(c >= 1)
                def _():
                    cwait_s(c - 1, buf_n, ss_n)
                cgather(c + 1, buf_n, sg_n)
            cwait_g(c, buf_c, sg_c)
            cscatter(c, buf_c, ss_c)

        @pl.when(c % 2 == 0)
        def _():
            stage(cb0, cb1, sg0, sg1, ss0, ss1)

        @pl.when(c % 2 == 1)
        def _():
            stage(cb1, cb0, sg1, sg0, ss1, ss0)
        return 0
    with jax.named_scope("p1_copy"):
        lax.fori_loop(0, ncc, copy_body, 0)

    @pl.when(ncc % 2 == 0)
    def _():
        cwait_s(ncc - 2, cb0, ss0)
        cwait_s(ncc - 1, cb1, ss1)

    @pl.when(ncc % 2 == 1)
    def _():
        cwait_s(ncc - 2, cb1, ss1)
        cwait_s(ncc - 1, cb0, ss0)

    tbase = base_r + ncc * CCH

    def tail_body(t, _):
        pltpu.async_copy(mem.at[pl.ds(tbase + t * CCT, CCT)],
                         cb0.at[pl.ds(0, CCT)], sg0).wait()
        pltpu.async_copy(cb0.at[pl.ds(0, CCT)],
                         new_mem.at[pl.ds(tbase + t * CCT, CCT)], ss0).wait()
        return 0
    with jax.named_scope("p1_tail"):
        lax.fori_loop(0, ntail, tail_body, 0)

    def init_body(i, _):
        last_b[pl.ds(i * 16, 16)] = jnp.full((16,), -1, jnp.int32)
        return 0
    with jax.named_scope("p2_init"):
        lax.fori_loop(0, PT // 16, init_body, 0)

    def scan_body(i, _):
        ids16 = ids_v[pl.ds(i * 16, 16)]
        mine = (ids16 >= base_r) & (ids16 < base_r + nrows)
        loc = ids16 - base_r
        bvec = i * 16 + iota
        plsc.store_scatter(last_b, [loc], bvec, mask=mine)
        rb = plsc.load_gather(last_b, [loc], mask=mine)
        need = mine & (rb < bvec)
        nf = jnp.sum(need.astype(jnp.int32))

        @pl.when(nf > 0)
        def _():
            def fix_body(c):
                rb2 = plsc.load_gather(last_b, [loc], mask=mine)
                need2 = mine & (rb2 < bvec)
                plsc.store_scatter(last_b, [loc], bvec, mask=need2)
                rb3 = plsc.load_gather(last_b, [loc], mask=mine)
                need3 = mine & (rb3 < bvec)
                return jnp.sum(need3.astype(jnp.int32))
            lax.while_loop(lambda c: c > 0, fix_body, nf)
        return 0

    gbase = wid * BPW
    SLAB = B // 16 // 4
    with jax.named_scope("p3_scan"):
        for k in range(4):
            gk = pltpu.async_copy(mem.at[ids_v.at[pl.ds(gbase + k * GCH, GCH)]],
                                  gbuf, sem_g)
            lax.fori_loop(k * SLAB, (k + 1) * SLAB, scan_body, 0)
            gk.wait()
            pltpu.sync_copy(gbuf, gathered.at[pl.ds(gbase + k * GCH, GCH)])

    cp_ts.wait()

    def tbl_body(i, cnt):
        lb = last_b[pl.ds(i * 16, 16)]
        m = lb >= 0
        t = plsc.load_gather(ts_v, [lb], mask=m)
        lu_v[pl.ds(i * 16, 16)] = jnp.where(m, t, jnp.float32(0.0))
        plsc.store_compressed(winb_flat.at[pl.ds(cnt, 16)], lb, mask=m)
        grow = base_r + i * 16 + iota
        plsc.store_compressed(dstr_flat.at[pl.ds(cnt, 16)], grow, mask=m)
        return cnt + jnp.sum(m.astype(jnp.int32))
    with jax.named_scope("p4_sweep"):
        cnt = lax.fori_loop(0, nvec, tbl_body, jnp.int32(0))

    @pl.when(cnt > 0)
    def _():
        lastix = jnp.full((16,), cnt - 1, jnp.int32)
        wpad = plsc.load_gather(winb_flat, [lastix])
        dpad = plsc.load_gather(dstr_flat, [lastix])
        for k in range(SCH // 16):
            winb_flat[pl.ds(cnt + k * 16, 16)] = wpad
            dstr_flat[pl.ds(cnt + k * 16, 16)] = dpad

    nch = (cnt + SCH - 1) // SCH

    def tr_body(j, _):
        v = dstr_flat[pl.ds(j * 16, 16)]
        dstr2d[j // 8, pl.ds((j % 8) * 16, 16)] = v
        return 0
    with jax.named_scope("p5_transpose"):
        lax.fori_loop(0, nch * (SCH // 16), tr_body, 0)

    @pl.when(jnp.logical_not(is_last))
    def _():
        pltpu.sync_copy(lu_v.at[pl.ds(0, RPW)], new_lu.at[pl.ds(base_r, RPW)])

    @pl.when(is_last)
    def _():
        pltpu.sync_copy(lu_v.at[pl.ds(0, LAST_ROWS)],
                        new_lu.at[pl.ds(base_r, LAST_ROWS)])

    def sc_body(c, _):
        pltpu.async_copy(values.at[winb_flat.at[pl.ds(c * SCH, SCH)]],
                         sbuf, sem_s).wait()
        pltpu.async_copy(sbuf, new_mem.at[dstr2d.at[c]], sem_s).wait()
        return 0
    with jax.named_scope("p6_scatter"):
        lax.fori_loop(0, nch, sc_body, 0)


def kernel(mem, values, timestamps, node_ids):
    mesh = plsc.VectorSubcoreMesh(core_axis_name="c", subcore_axis_name="s")
    out = pl.kernel(
        _body,
        out_type=(
            jax.ShapeDtypeStruct((B, D), jnp.float32),
            jax.ShapeDtypeStruct((M, D), jnp.float32),
            jax.ShapeDtypeStruct((M,), jnp.float32),
        ),
        mesh=mesh,
        compiler_params=pltpu.CompilerParams(needs_layout_passes=False),
        scratch_types=[
            pltpu.VMEM((B,), jnp.int32),
            pltpu.VMEM((B,), jnp.float32),
            pltpu.VMEM((PT,), jnp.int32),
            pltpu.VMEM((PT,), jnp.float32),
            pltpu.VMEM((LIST_CAP,), jnp.int32),
            pltpu.VMEM((LIST_CAP,), jnp.int32),
            pltpu.VMEM((NCH_MAX, SCH), jnp.int32),
            pltpu.VMEM((GCH, D), jnp.float32),
            pltpu.VMEM((SCH, D), jnp.float32),
            pltpu.VMEM((CCH, D), jnp.float32),
            pltpu.VMEM((CCH, D), jnp.float32),
            pltpu.SemaphoreType.DMA,
            pltpu.SemaphoreType.DMA,
            pltpu.SemaphoreType.DMA,
            pltpu.SemaphoreType.DMA,
            pltpu.SemaphoreType.DMA,
            pltpu.SemaphoreType.DMA,
            pltpu.SemaphoreType.DMA,
        ],
    )(mem, values, timestamps, node_ids)
    return out


# speedup vs baseline: 17.3110x; 1.3718x over previous
import jax
import jax.numpy as jnp
from jax import lax
from jax.experimental import pallas as pl
from jax.experimental.pallas import tpu as pltpu
from jax.experimental.pallas import tpu_sc as plsc

M = 100000
D = 128
B = 16384
NC = 2
NS = 16
NW = NC * NS

RPW = 3136
LAST_ROWS = M - (NW - 1) * RPW
PT = RPW
BPW = B // NW
GCH = 128
SCH = 128
CCH = 128
CCT = 32
NSV = B // 16
NCH_MAX = (RPW + SCH - 1) // SCH
LIST_CAP = RPW + 2 * SCH


def _body(mem, values, ts, ids, gathered, new_mem, new_lu,
          ids_v, ts_v, last_b, lu_v, winb_flat, dstr_flat, dstr2d,
          gbuf, sbuf, cb0, cb1,
          sem_t, sem_g, sg0, sg1, ss0, ss1):
    wid = lax.axis_index("s") * NC + lax.axis_index("c")
    base_r = wid * RPW
    is_last = wid == NW - 1
    nrows = jnp.where(is_last, LAST_ROWS, RPW).astype(jnp.int32)
    nvec = nrows // 16
    iota = lax.broadcasted_iota(jnp.int32, (16,), 0)

    ncc = nrows // CCH
    ntail = (nrows - ncc * CCH) // CCT

    def cgather(c, buf, sem):
        pltpu.async_copy(mem.at[pl.ds(base_r + c * CCH, CCH)], buf, sem)

    def cscatter(c, buf, sem):
        pltpu.async_copy(buf, new_mem.at[pl.ds(base_r + c * CCH, CCH)], sem)

    def cwait_g(c, buf, sem):
        pltpu.make_async_copy(mem.at[pl.ds(base_r + c * CCH, CCH)], buf, sem).wait()

    def cwait_s(c, buf, sem):
        pltpu.make_async_copy(buf, new_mem.at[pl.ds(base_r + c * CCH, CCH)],
                              sem).wait()

    cgather(0, cb0, sg0)
    cp_ts = pltpu.async_copy(ts, ts_v, sem_t)
    pltpu.sync_copy(ids, ids_v)

    def init_body(i, _):
        last_b[pl.ds(i * 16, 16)] = jnp.full((16,), -1, jnp.int32)
        return 0
    lax.fori_loop(0, PT // 16, init_body, 0)

    def scan_body(i, _):
        ids16 = ids_v[pl.ds(i * 16, 16)]
        mine = (ids16 >= base_r) & (ids16 < base_r + nrows)
        plsc.store_scatter(last_b, [ids16 - base_r], i * 16 + iota, mask=mine)
        return 0

    def copy_body(c, _):
        @pl.when(c % 2 == 0)
        def _():
            @pl.when(c + 1 < ncc)
            def _():
                @pl.when(c >= 1)
                def _():
                    cwait_s(c - 1, cb1, ss1)
                cgather(c + 1, cb1, sg1)

        @pl.when(c % 2 == 1)
        def _():
            @pl.when(c + 1 < ncc)
            def _():
                cwait_s(c - 1, cb0, ss0)
                cgather(c + 1, cb0, sg0)

        lax.fori_loop(c * NSV // ncc, (c + 1) * NSV // ncc, scan_body, 0)

        @pl.when(c % 2 == 0)
        def _():
            cwait_g(c, cb0, sg0)
            cscatter(c, cb0, ss0)

        @pl.when(c % 2 == 1)
        def _():
            cwait_g(c, cb1, sg1)
            cscatter(c, cb1, ss1)
        return 0
    lax.fori_loop(0, ncc, copy_body, 0)

    @pl.when(ncc % 2 == 0)
    def _():
        cwait_s(ncc - 2, cb0, ss0)
        cwait_s(ncc - 1, cb1, ss1)

    @pl.when(ncc % 2 == 1)
    def _():
        cwait_s(ncc - 2, cb1, ss1)
        cwait_s(ncc - 1, cb0, ss0)

    tbase = base_r + ncc * CCH

    def tail_body(t, _):
        pltpu.async_copy(mem.at[pl.ds(tbase + t * CCT, CCT)],
                         cb0.at[pl.ds(0, CCT)], sg0).wait()
        pltpu.async_copy(cb0.at[pl.ds(0, CCT)],
                         new_mem.at[pl.ds(tbase + t * CCT, CCT)], ss0).wait()
        return 0
    lax.fori_loop(0, ntail, tail_body, 0)

    gbase = wid * BPW

    def gfire(k, buf, sem):
        pltpu.async_copy(mem.at[ids_v.at[pl.ds(gbase + k * GCH, GCH)]], buf, sem)

    def gwait(k, buf, sem):
        pltpu.make_async_copy(mem.at[ids_v.at[pl.ds(gbase + k * GCH, GCH)]],
                              buf, sem).wait()

    gfire(0, gbuf, sg0)
    gfire(1, sbuf, sg1)

    cp_ts.wait()

    def tbl_body(i, cnt):
        lb = last_b[pl.ds(i * 16, 16)]
        m = lb >= 0
        t = plsc.load_gather(ts_v, [lb], mask=m)
        lu_v[pl.ds(i * 16, 16)] = jnp.where(m, t, jnp.float32(0.0))
        plsc.store_compressed(winb_flat.at[pl.ds(cnt, 16)], lb, mask=m)
        grow = base_r + i * 16 + iota
        plsc.store_compressed(dstr_flat.at[pl.ds(cnt, 16)], grow, mask=m)
        return cnt + jnp.sum(m.astype(jnp.int32))
    cnt = lax.fori_loop(0, nvec, tbl_body, jnp.int32(0))

    gwait(0, gbuf, sg0)
    pltpu.sync_copy(gbuf, gathered.at[pl.ds(gbase, GCH)])
    gfire(2, gbuf, sg0)
    gwait(1, sbuf, sg1)
    pltpu.sync_copy(sbuf, gathered.at[pl.ds(gbase + GCH, GCH)])
    gfire(3, sbuf, sg1)

    @pl.when(cnt > 0)
    def _():
        lastix = jnp.full((16,), cnt - 1, jnp.int32)
        wpad = plsc.load_gather(winb_flat, [lastix])
        dpad = plsc.load_gather(dstr_flat, [lastix])
        for k in range(SCH // 16):
            winb_flat[pl.ds(cnt + k * 16, 16)] = wpad
            dstr_flat[pl.ds(cnt + k * 16, 16)] = dpad

    nch = (cnt + SCH - 1) // SCH

    def tr_body(j, _):
        v = dstr_flat[pl.ds(j * 16, 16)]
        dstr2d[j // 8, pl.ds((j % 8) * 16, 16)] = v
        return 0
    lax.fori_loop(0, nch * (SCH // 16), tr_body, 0)

    @pl.when(jnp.logical_not(is_last))
    def _():
        pltpu.sync_copy(lu_v.at[pl.ds(0, RPW)], new_lu.at[pl.ds(base_r, RPW)])

    @pl.when(is_last)
    def _():
        pltpu.sync_copy(lu_v.at[pl.ds(0, LAST_ROWS)],
                        new_lu.at[pl.ds(base_r, LAST_ROWS)])

    gwait(2, gbuf, sg0)
    pltpu.sync_copy(gbuf, gathered.at[pl.ds(gbase + 2 * GCH, GCH)])
    gwait(3, sbuf, sg1)
    pltpu.sync_copy(sbuf, gathered.at[pl.ds(gbase + 3 * GCH, GCH)])

    def vg(c, buf, sem):
        pltpu.async_copy(values.at[winb_flat.at[pl.ds(c * SCH, SCH)]], buf, sem)

    def vgw(c, buf, sem):
        pltpu.make_async_copy(values.at[winb_flat.at[pl.ds(c * SCH, SCH)]],
                              buf, sem).wait()

    def rs(c, buf, sem):
        pltpu.async_copy(buf, new_mem.at[dstr2d.at[c]], sem)

    def rsw(c, buf, sem):
        pltpu.make_async_copy(buf, new_mem.at[dstr2d.at[c]], sem).wait()

    @pl.when(nch > 0)
    def _():
        vg(0, gbuf, sg0)

        def sc_body(c, _):
            @pl.when(c % 2 == 0)
            def _():
                @pl.when(c + 1 < nch)
                def _():
                    @pl.when(c >= 1)
                    def _():
                        rsw(c - 1, sbuf, ss1)
                    vg(c + 1, sbuf, sg1)
                vgw(c, gbuf, sg0)
                rs(c, gbuf, ss0)

            @pl.when(c % 2 == 1)
            def _():
                @pl.when(c + 1 < nch)
                def _():
                    rsw(c - 1, gbuf, ss0)
                    vg(c + 1, gbuf, sg0)
                vgw(c, sbuf, sg1)
                rs(c, sbuf, ss1)
            return 0
        lax.fori_loop(0, nch, sc_body, 0)

        @pl.when(nch == 1)
        def _():
            rsw(0, gbuf, ss0)

        @pl.when((nch > 1) & (nch % 2 == 0))
        def _():
            rsw(nch - 2, gbuf, ss0)
            rsw(nch - 1, sbuf, ss1)

        @pl.when((nch > 1) & (nch % 2 == 1))
        def _():
            rsw(nch - 2, sbuf, ss1)
            rsw(nch - 1, gbuf, ss0)


def kernel(mem, values, timestamps, node_ids):
    mesh = plsc.VectorSubcoreMesh(core_axis_name="c", subcore_axis_name="s")
    out = pl.kernel(
        _body,
        out_type=(
            jax.ShapeDtypeStruct((B, D), jnp.float32),
            jax.ShapeDtypeStruct((M, D), jnp.float32),
            jax.ShapeDtypeStruct((M,), jnp.float32),
        ),
        mesh=mesh,
        compiler_params=pltpu.CompilerParams(needs_layout_passes=False),
        scratch_types=[
            pltpu.VMEM((B,), jnp.int32),
            pltpu.VMEM((B,), jnp.float32),
            pltpu.VMEM((PT,), jnp.int32),
            pltpu.VMEM((PT,), jnp.float32),
            pltpu.VMEM((LIST_CAP,), jnp.int32),
            pltpu.VMEM((LIST_CAP,), jnp.int32),
            pltpu.VMEM((NCH_MAX, SCH), jnp.int32),
            pltpu.VMEM((GCH, D), jnp.float32),
            pltpu.VMEM((SCH, D), jnp.float32),
            pltpu.VMEM((CCH, D), jnp.float32),
            pltpu.VMEM((CCH, D), jnp.float32),
            pltpu.SemaphoreType.DMA,
            pltpu.SemaphoreType.DMA,
            pltpu.SemaphoreType.DMA,
            pltpu.SemaphoreType.DMA,
            pltpu.SemaphoreType.DMA,
            pltpu.SemaphoreType.DMA,
        ],
    )(mem, values, timestamps, node_ids)
    return out


# speedup vs baseline: 17.3517x; 1.0024x over previous
import jax
import jax.numpy as jnp
from jax import lax
from jax.experimental import pallas as pl
from jax.experimental.pallas import tpu as pltpu
from jax.experimental.pallas import tpu_sc as plsc

M = 100000
D = 128
B = 16384
NC = 2
NS = 16
NW = NC * NS

RPW = 3136
LAST_ROWS = M - (NW - 1) * RPW
PT = RPW
BPW = B // NW
GCH = 128
SCH = 128
CCH = 128
CCT = 32
NSV = B // 16
NCH_MAX = (RPW + SCH - 1) // SCH
LIST_CAP = RPW + 2 * SCH


def _body(mem, values, ts, ids, gathered, new_mem, new_lu,
          ids_v, ts_v, last_b, lu_v, winb_flat, dstr_flat, dstr2d,
          gbuf, sbuf, cb0, cb1,
          sem_t, sem_g, sg0, sg1, ss0, ss1):
    wid = lax.axis_index("s") * NC + lax.axis_index("c")
    base_r = wid * RPW
    is_last = wid == NW - 1
    nrows = jnp.where(is_last, LAST_ROWS, RPW).astype(jnp.int32)
    nvec = nrows // 16
    iota = lax.broadcasted_iota(jnp.int32, (16,), 0)

    ncc = nrows // CCH
    ntail = (nrows - ncc * CCH) // CCT

    def cgather(c, buf, sem):
        pltpu.async_copy(mem.at[pl.ds(base_r + c * CCH, CCH)], buf, sem)

    def cscatter(c, buf, sem):
        pltpu.async_copy(buf, new_mem.at[pl.ds(base_r + c * CCH, CCH)], sem)

    def cwait_g(c, buf, sem):
        pltpu.make_async_copy(mem.at[pl.ds(base_r + c * CCH, CCH)], buf, sem).wait()

    def cwait_s(c, buf, sem):
        pltpu.make_async_copy(buf, new_mem.at[pl.ds(base_r + c * CCH, CCH)],
                              sem).wait()

    cgather(0, cb0, sg0)
    cp_ts = pltpu.async_copy(ts, ts_v, sem_t)
    pltpu.sync_copy(ids, ids_v)

    def init_body(i, _):
        last_b[pl.ds(i * 16, 16)] = jnp.full((16,), -1, jnp.int32)
        return 0
    lax.fori_loop(0, PT // 16, init_body, 0)

    def scan_body(i, _):
        ids16 = ids_v[pl.ds(i * 16, 16)]
        mine = (ids16 >= base_r) & (ids16 < base_r + nrows)
        plsc.store_scatter(last_b, [ids16 - base_r], i * 16 + iota, mask=mine)
        return 0

    def copy_body(c, _):
        @pl.when(c % 2 == 0)
        def _():
            @pl.when(c + 1 < ncc)
            def _():
                @pl.when(c >= 1)
                def _():
                    cwait_s(c - 1, cb1, ss1)
                cgather(c + 1, cb1, sg1)

        @pl.when(c % 2 == 1)
        def _():
            @pl.when(c + 1 < ncc)
            def _():
                cwait_s(c - 1, cb0, ss0)
                cgather(c + 1, cb0, sg0)

        lax.fori_loop(c * NSV // ncc, (c + 1) * NSV // ncc, scan_body, 0)

        @pl.when(c % 2 == 0)
        def _():
            cwait_g(c, cb0, sg0)
            cscatter(c, cb0, ss0)

        @pl.when(c % 2 == 1)
        def _():
            cwait_g(c, cb1, sg1)
            cscatter(c, cb1, ss1)
        return 0
    with jax.named_scope("p1_ring"):
        lax.fori_loop(0, ncc, copy_body, 0)

    @pl.when(ncc % 2 == 0)
    def _():
        cwait_s(ncc - 2, cb0, ss0)
        cwait_s(ncc - 1, cb1, ss1)

    @pl.when(ncc % 2 == 1)
    def _():
        cwait_s(ncc - 2, cb1, ss1)
        cwait_s(ncc - 1, cb0, ss0)

    tbase = base_r + ncc * CCH

    def tail_body(t, _):
        pltpu.async_copy(mem.at[pl.ds(tbase + t * CCT, CCT)],
                         cb0.at[pl.ds(0, CCT)], sg0).wait()
        pltpu.async_copy(cb0.at[pl.ds(0, CCT)],
                         new_mem.at[pl.ds(tbase + t * CCT, CCT)], ss0).wait()
        return 0
    with jax.named_scope("p1_tail"):
        lax.fori_loop(0, ntail, tail_body, 0)

    gbase = wid * BPW

    def gfire(k, buf, sem):
        pltpu.async_copy(mem.at[ids_v.at[pl.ds(gbase + k * GCH, GCH)]], buf, sem)

    def gwait(k, buf, sem):
        pltpu.make_async_copy(mem.at[ids_v.at[pl.ds(gbase + k * GCH, GCH)]],
                              buf, sem).wait()

    gfire(0, gbuf, sg0)
    gfire(1, sbuf, sg1)

    cp_ts.wait()

    def tbl_body(i, cnt):
        lb = last_b[pl.ds(i * 16, 16)]
        m = lb >= 0
        t = plsc.load_gather(ts_v, [lb], mask=m)
        lu_v[pl.ds(i * 16, 16)] = jnp.where(m, t, jnp.float32(0.0))
        plsc.store_compressed(winb_flat.at[pl.ds(cnt, 16)], lb, mask=m)
        grow = base_r + i * 16 + iota
        plsc.store_compressed(dstr_flat.at[pl.ds(cnt, 16)], grow, mask=m)
        return cnt + jnp.sum(m.astype(jnp.int32))
    with jax.named_scope("p4_sweep"):
        cnt = lax.fori_loop(0, nvec, tbl_body, jnp.int32(0))

    gwait(0, gbuf, sg0)
    pltpu.sync_copy(gbuf, gathered.at[pl.ds(gbase, GCH)])
    gfire(2, gbuf, sg0)
    gwait(1, sbuf, sg1)
    pltpu.sync_copy(sbuf, gathered.at[pl.ds(gbase + GCH, GCH)])
    gfire(3, sbuf, sg1)

    @pl.when(cnt > 0)
    def _():
        lastix = jnp.full((16,), cnt - 1, jnp.int32)
        wpad = plsc.load_gather(winb_flat, [lastix])
        dpad = plsc.load_gather(dstr_flat, [lastix])
        for k in range(SCH // 16):
            winb_flat[pl.ds(cnt + k * 16, 16)] = wpad
            dstr_flat[pl.ds(cnt + k * 16, 16)] = dpad

    nch = (cnt + SCH - 1) // SCH

    def tr_body(j, _):
        v = dstr_flat[pl.ds(j * 16, 16)]
        dstr2d[j // 8, pl.ds((j % 8) * 16, 16)] = v
        return 0
    lax.fori_loop(0, nch * (SCH // 16), tr_body, 0)

    @pl.when(jnp.logical_not(is_last))
    def _():
        pltpu.sync_copy(lu_v.at[pl.ds(0, RPW)], new_lu.at[pl.ds(base_r, RPW)])

    @pl.when(is_last)
    def _():
        pltpu.sync_copy(lu_v.at[pl.ds(0, LAST_ROWS)],
                        new_lu.at[pl.ds(base_r, LAST_ROWS)])

    with jax.named_scope("p5_gout"):
        gwait(2, gbuf, sg0)
        pltpu.sync_copy(gbuf, gathered.at[pl.ds(gbase + 2 * GCH, GCH)])
        gwait(3, sbuf, sg1)
        pltpu.sync_copy(sbuf, gathered.at[pl.ds(gbase + 3 * GCH, GCH)])

    def vg(c, buf, sem):
        pltpu.async_copy(values.at[winb_flat.at[pl.ds(c * SCH, SCH)]], buf, sem)

    def vgw(c, buf, sem):
        pltpu.make_async_copy(values.at[winb_flat.at[pl.ds(c * SCH, SCH)]],
                              buf, sem).wait()

    def rs(c, buf, sem):
        pltpu.async_copy(buf, new_mem.at[dstr2d.at[c]], sem)

    def rsw(c, buf, sem):
        pltpu.make_async_copy(buf, new_mem.at[dstr2d.at[c]], sem).wait()

    @pl.when(nch > 0)
    def _():
        vg(0, gbuf, sg0)

        def sc_body(c, _):
            @pl.when(c % 2 == 0)
            def _():
                @pl.when(c + 1 < nch)
                def _():
                    @pl.when(c >= 1)
                    def _():
                        rsw(c - 1, sbuf, ss1)
                    vg(c + 1, sbuf, sg1)
                vgw(c, gbuf, sg0)
                rs(c, gbuf, ss0)

            @pl.when(c % 2 == 1)
            def _():
                @pl.when(c + 1 < nch)
                def _():
                    rsw(c - 1, gbuf, ss0)
                    vg(c + 1, gbuf, sg0)
                vgw(c, sbuf, sg1)
                rs(c, sbuf, ss1)
            return 0
        with jax.named_scope("p6_scatter"):
            lax.fori_loop(0, nch, sc_body, 0)

        @pl.when(nch == 1)
        def _():
            rsw(0, gbuf, ss0)

        @pl.when((nch > 1) & (nch % 2 == 0))
        def _():
            rsw(nch - 2, gbuf, ss0)
            rsw(nch - 1, sbuf, ss1)

        @pl.when((nch > 1) & (nch % 2 == 1))
        def _():
            rsw(nch - 2, sbuf, ss1)
            rsw(nch - 1, gbuf, ss0)


def kernel(mem, values, timestamps, node_ids):
    mesh = plsc.VectorSubcoreMesh(core_axis_name="c", subcore_axis_name="s")
    out = pl.kernel(
        _body,
        out_type=(
            jax.ShapeDtypeStruct((B, D), jnp.float32),
            jax.ShapeDtypeStruct((M, D), jnp.float32),
            jax.ShapeDtypeStruct((M,), jnp.float32),
        ),
        mesh=mesh,
        compiler_params=pltpu.CompilerParams(needs_layout_passes=False),
        scratch_types=[
            pltpu.VMEM((B,), jnp.int32),
            pltpu.VMEM((B,), jnp.float32),
            pltpu.VMEM((PT,), jnp.int32),
            pltpu.VMEM((PT,), jnp.float32),
            pltpu.VMEM((LIST_CAP,), jnp.int32),
            pltpu.VMEM((LIST_CAP,), jnp.int32),
            pltpu.VMEM((NCH_MAX, SCH), jnp.int32),
            pltpu.VMEM((GCH, D), jnp.float32),
            pltpu.VMEM((SCH, D), jnp.float32),
            pltpu.VMEM((CCH, D), jnp.float32),
            pltpu.VMEM((CCH, D), jnp.float32),
            pltpu.SemaphoreType.DMA,
            pltpu.SemaphoreType.DMA,
            pltpu.SemaphoreType.DMA,
            pltpu.SemaphoreType.DMA,
            pltpu.SemaphoreType.DMA,
            pltpu.SemaphoreType.DMA,
        ],
    )(mem, values, timestamps, node_ids)
    return out
